# baseline (speedup 1.0000x reference)
import jax
import jax.numpy as jnp
from jax import lax
from jax.experimental import pallas as pl
from jax.experimental.pallas import tpu as pltpu
from jax.experimental.pallas import tpu_sc as plsc

B, S, D = 4, 2048, 1024
B_SC = 2
NC, NS = 2, 16
NW = NC * NS
S_PER_W = S // NW
R = 16
SUBS = S_PER_W // R
NCHUNKS = B_SC * SUBS
LANES = 16
VREGS_PER_ROW = D // LANES
VREGS_PER_CHUNK = R * VREGS_PER_ROW
TC_BS = 256


def _sc_body(x_hbm, pt_hbm, out_hbm, pe_buf, xb0, xb1, si0, si1, so0, so1):
    wid = lax.axis_index("s") * NC + lax.axis_index("c")
    s0 = wid * S_PER_W
    bufs = (xb0, xb1)
    in_sems = (si0, si1)
    out_sems = (so0, so1)

    pltpu.sync_copy(pt_hbm.at[pl.ds(s0, S_PER_W)], pe_buf)

    def in_copy(i):
        b, sub = divmod(i, SUBS)
        return pltpu.make_async_copy(
            x_hbm.at[b, pl.ds(s0 + sub * R, R)], bufs[i % 2], in_sems[i % 2])

    def out_copy(i):
        b, sub = divmod(i, SUBS)
        return pltpu.make_async_copy(
            bufs[i % 2], out_hbm.at[b, pl.ds(s0 + sub * R, R)],
            out_sems[i % 2])

    in_copy(0).start()
    for i in range(NCHUNKS):
        if i + 1 < NCHUNKS:
            if i >= 1:
                out_copy(i - 1).wait()
            in_copy(i + 1).start()
        in_copy(i).wait()

        buf = bufs[i % 2]
        row_base = (i % SUBS) * R

        @plsc.parallel_loop(0, VREGS_PER_CHUNK, unroll=8)
        def _(v):
            r = v >> 6
            coff = (v & (VREGS_PER_ROW - 1)) * LANES
            plsc.addupdate(
                buf.at[r, pl.ds(coff, LANES)],
                pe_buf[row_base + r, pl.ds(coff, LANES)],
            )

        out_copy(i).start()
    out_copy(NCHUNKS - 2).wait()
    out_copy(NCHUNKS - 1).wait()


def _tc_body(x_ref, pe_ref, o_ref):
    o_ref[...] = x_ref[...] + pe_ref[...][None]


@jax.jit
def kernel(x, pos_table):
    mesh = plsc.VectorSubcoreMesh(core_axis_name="c", subcore_axis_name="s")
    sc_out = pl.kernel(
        _sc_body,
        out_type=jax.ShapeDtypeStruct((B_SC, S, D), jnp.float32),
        mesh=mesh,
        scratch_types=[
            pltpu.VMEM((S_PER_W, D), jnp.float32),
            pltpu.VMEM((R, D), jnp.float32),
            pltpu.VMEM((R, D), jnp.float32),
            pltpu.SemaphoreType.DMA,
            pltpu.SemaphoreType.DMA,
            pltpu.SemaphoreType.DMA,
            pltpu.SemaphoreType.DMA,
        ],
    )(x, pos_table)

    tc_out = pl.pallas_call(
        _tc_body,
        grid=(S // TC_BS, B - B_SC),
        in_specs=[
            pl.BlockSpec((1, TC_BS, D), lambda i, b: (b + B_SC, i, 0)),
            pl.BlockSpec((TC_BS, D), lambda i, b: (i, 0)),
        ],
        out_specs=pl.BlockSpec((1, TC_BS, D), lambda i, b: (b, i, 0)),
        out_shape=jax.ShapeDtypeStruct((B - B_SC, S, D), jnp.float32),
    )(x, pos_table)

    return jnp.concatenate([sc_out, tc_out], axis=0)


# speedup vs baseline: 1.2421x; 1.2421x over previous
import jax
import jax.numpy as jnp
from jax import lax
from jax.experimental import pallas as pl
from jax.experimental.pallas import tpu as pltpu
from jax.experimental.pallas import tpu_sc as plsc

B, S, D = 4, 2048, 1024
NC, NS = 2, 16
NW = NC * NS
S_PER_W = S // NW
R = 16
SUBS = S_PER_W // R
NCHUNKS = B * SUBS
LANES = 16
VREGS_PER_ROW = D // LANES
VREGS_PER_CHUNK = R * VREGS_PER_ROW


def _sc_body(x_hbm, pt_hbm, out_hbm, pe_buf, xb0, xb1, out_sp,
             si0, si1, so0, sts, ssh):
    cid = lax.axis_index("c")
    sid = lax.axis_index("s")
    wid = sid * NC + cid
    s0 = wid * S_PER_W
    bufs = (xb0, xb1)
    in_sems = (si0, si1)

    pltpu.sync_copy(pt_hbm.at[pl.ds(s0, S_PER_W)], pe_buf)

    def loc(i):
        b, sub = divmod(i, SUBS)
        return b, s0 + sub * R

    def in_copy(i):
        b, r0 = loc(i)
        return pltpu.make_async_copy(
            x_hbm.at[b, pl.ds(r0, R)], bufs[i % 2], in_sems[i % 2])

    def out_direct(i):
        b, r0 = loc(i)
        return pltpu.make_async_copy(
            bufs[i % 2], out_hbm.at[b, pl.ds(r0, R)], so0)

    def tile_sp(i):
        return pltpu.make_async_copy(bufs[i % 2], out_sp.at[sid], sts)

    def sp_hbm(i):
        b, r0 = loc(i)
        return pltpu.make_async_copy(
            out_sp.at[sid], out_hbm.at[b, pl.ds(r0, R)], ssh)

    in_copy(0).start()
    for i in range(NCHUNKS):
        if i >= 1 and (i - 1) % 2 == 1:
            tile_sp(i - 1).wait()
            sp_hbm(i - 1).start()
        if i + 1 < NCHUNKS:
            if i >= 1 and (i - 1) % 2 == 0:
                out_direct(i - 1).wait()
            in_copy(i + 1).start()
        in_copy(i).wait()

        buf = bufs[i % 2]
        row_base = (i % SUBS) * R

        @plsc.parallel_loop(0, VREGS_PER_CHUNK, unroll=8)
        def _(v):
            r = v >> 6
            coff = (v & (VREGS_PER_ROW - 1)) * LANES
            plsc.addupdate(
                buf.at[r, pl.ds(coff, LANES)],
                pe_buf[row_base + r, pl.ds(coff, LANES)],
            )

        if i % 2 == 0:
            out_direct(i).start()
        else:
            if i >= 3:
                sp_hbm(i - 2).wait()
            tile_sp(i).start()

    tile_sp(NCHUNKS - 1).wait()
    sp_hbm(NCHUNKS - 1).start()
    sp_hbm(NCHUNKS - 1).wait()
    out_direct(NCHUNKS - 2).wait()


@jax.jit
def kernel(x, pos_table):
    mesh = plsc.VectorSubcoreMesh(core_axis_name="c", subcore_axis_name="s")
    return pl.kernel(
        _sc_body,
        out_type=jax.ShapeDtypeStruct((B, S, D), jnp.float32),
        mesh=mesh,
        scratch_types=[
            pltpu.VMEM((S_PER_W, D), jnp.float32),
            pltpu.VMEM((R, D), jnp.float32),
            pltpu.VMEM((R, D), jnp.float32),
            pltpu.VMEM_SHARED((NS, R, D), jnp.float32),
            pltpu.SemaphoreType.DMA,
            pltpu.SemaphoreType.DMA,
            pltpu.SemaphoreType.DMA,
            pltpu.SemaphoreType.DMA,
            pltpu.SemaphoreType.DMA,
        ],
    )(x, pos_table)


# speedup vs baseline: 1.3372x; 1.0765x over previous
import jax
import jax.numpy as jnp
from jax import lax
from jax.experimental import pallas as pl
from jax.experimental.pallas import tpu as pltpu
from jax.experimental.pallas import tpu_sc as plsc

B, S, D = 4, 2048, 1024
NC, NS = 2, 16
NW = NC * NS
S_PER_W = S // NW
R = 16
SUBS = S_PER_W // R
NCHUNKS = B * SUBS
NBUF = 3
LANES = 16
VREGS_PER_ROW = D // LANES
VREGS_PER_CHUNK = R * VREGS_PER_ROW


def _sc_body(x_hbm, pt_hbm, out_hbm, pe_buf, xb0, xb1, xb2,
             si0, si1, si2, so0, so1, so2, spe0, spe1, spe2, spe3):
    cid = lax.axis_index("c")
    sid = lax.axis_index("s")
    wid = sid * NC + cid
    s0 = wid * S_PER_W
    bufs = (xb0, xb1, xb2)
    in_sems = (si0, si1, si2)
    out_sems = (so0, so1, so2)

    def loc(i):
        b, sub = divmod(i, SUBS)
        return b, s0 + sub * R

    def in_copy(i):
        b, r0 = loc(i)
        return pltpu.make_async_copy(
            x_hbm.at[b, pl.ds(r0, R)], bufs[i % NBUF], in_sems[i % NBUF])

    def out_copy(i):
        b, r0 = loc(i)
        return pltpu.make_async_copy(
            bufs[i % NBUF], out_hbm.at[b, pl.ds(r0, R)], out_sems[i % NBUF])

    pe_sems = (spe0, spe1, spe2, spe3)

    def pe_copy(sub):
        return pltpu.make_async_copy(
            pt_hbm.at[pl.ds(s0 + sub * R, R)],
            pe_buf.at[pl.ds(sub * R, R)], pe_sems[sub])

    in_copy(0).start()
    in_copy(1).start()
    for sub in range(SUBS):
        pe_copy(sub).start()

    for i in range(NCHUNKS):
        if i + 2 < NCHUNKS:
            if i >= 1:
                out_copy(i - 1).wait()
            in_copy(i + 2).start()
        if i < SUBS:
            pe_copy(i).wait()
        in_copy(i).wait()

        buf = bufs[i % NBUF]
        row_base = (i % SUBS) * R

        @plsc.parallel_loop(0, VREGS_PER_CHUNK, unroll=8)
        def _(v):
            r = v >> 6
            coff = (v & (VREGS_PER_ROW - 1)) * LANES
            plsc.addupdate(
                buf.at[r, pl.ds(coff, LANES)],
                pe_buf[row_base + r, pl.ds(coff, LANES)],
            )

        out_copy(i).start()
    out_copy(NCHUNKS - 3).wait()
    out_copy(NCHUNKS - 2).wait()
    out_copy(NCHUNKS - 1).wait()


@jax.jit
def kernel(x, pos_table):
    mesh = plsc.VectorSubcoreMesh(core_axis_name="c", subcore_axis_name="s")
    return pl.kernel(
        _sc_body,
        out_type=jax.ShapeDtypeStruct((B, S, D), jnp.float32),
        mesh=mesh,
        scratch_types=[
            pltpu.VMEM((S_PER_W, D), jnp.float32),
            pltpu.VMEM((R, D), jnp.float32),
            pltpu.VMEM((R, D), jnp.float32),
            pltpu.VMEM((R, D), jnp.float32),
            pltpu.SemaphoreType.DMA,
            pltpu.SemaphoreType.DMA,
            pltpu.SemaphoreType.DMA,
            pltpu.SemaphoreType.DMA,
            pltpu.SemaphoreType.DMA,
            pltpu.SemaphoreType.DMA,
            pltpu.SemaphoreType.DMA,
            pltpu.SemaphoreType.DMA,
            pltpu.SemaphoreType.DMA,
            pltpu.SemaphoreType.DMA,
        ],
    )(x, pos_table)
